# baseline (speedup 1.0000x reference)
import functools

import jax
import jax.numpy as jnp
from jax import lax
from jax.experimental import pallas as pl
from jax.experimental.pallas import tpu as pltpu
from jax.experimental.pallas import tpu_sc as plsc

_NC = 2
_NS = 16
_L = 16

_BS = 16
_H = 32
_W = 32
_NF = 128


def _lookup_body(row_hbm, col_hbm, re_out, ce_out, idxv, rows_v, sem):
    w = lax.axis_index("s") * _NC + lax.axis_index("c")
    seg = (w % 2) * _L
    idxv[pl.ds(0, _L)] = lax.iota(jnp.int32, _L) + seg

    @pl.when(w // 2 == 0)
    def _():
        pltpu.async_copy(col_hbm.at[idxv], rows_v, sem).wait()
        pltpu.sync_copy(rows_v, ce_out.at[pl.ds(seg, _L)])

    @pl.when(w // 2 == 1)
    def _():
        pltpu.async_copy(row_hbm.at[idxv], rows_v, sem).wait()
        pltpu.sync_copy(rows_v, re_out.at[pl.ds(seg, _L)])


_lookup_sc = functools.partial(
    pl.kernel,
    out_type=(
        jax.ShapeDtypeStruct((_H, _NF), jnp.float32),
        jax.ShapeDtypeStruct((_W, _NF), jnp.float32),
    ),
    mesh=plsc.VectorSubcoreMesh(core_axis_name="c", subcore_axis_name="s"),
    scratch_types=[
        pltpu.VMEM((_L,), jnp.int32),
        pltpu.VMEM((_L, _NF), jnp.float32),
        pltpu.SemaphoreType.DMA,
    ],
    compiler_params=pltpu.CompilerParams(needs_layout_passes=False),
)(_lookup_body)


def _bcast_body(ce_ref, re_ref, out_ref):
    out_ref[0, :, :, 0:_NF] = jnp.broadcast_to(
        ce_ref[...][None, :, :], (_H, _W, _NF)
    )
    out_ref[0, :, :, _NF : 2 * _NF] = jnp.broadcast_to(
        re_ref[...][:, None, :], (_H, _W, _NF)
    )


def kernel(mask, row_embed, col_embed):
    bs, h, w = mask.shape
    re_lk, ce_lk = _lookup_sc(row_embed, col_embed)
    out = pl.pallas_call(
        _bcast_body,
        grid=(_BS,),
        in_specs=[
            pl.BlockSpec((_W, _NF), lambda b: (0, 0)),
            pl.BlockSpec((_H, _NF), lambda b: (0, 0)),
        ],
        out_specs=pl.BlockSpec((1, _H, _W, 2 * _NF), lambda b: (b, 0, 0, 0)),
        out_shape=jax.ShapeDtypeStruct((_BS, _H, _W, 2 * _NF), jnp.float32),
    )(ce_lk, re_lk)
    return jnp.transpose(out, (0, 3, 1, 2))
